# Initial kernel scaffold; baseline (speedup 1.0000x reference)
import jax
import jax.numpy as jnp
from jax.experimental import pallas as pl


def kernel(embed, b0, w1, b1, w2, b2, edge_index_0, edge_index_1, edge_index_2):
    raise NotImplementedError("write your pallas kernel here")


# speedup vs baseline: 5.2811x; 5.2811x over previous
import functools

import jax
import jax.numpy as jnp
from jax import lax
from jax.experimental import pallas as pl
from jax.experimental.pallas import tpu as pltpu
from jax.experimental.pallas import tpu_sc as plsc

N = 10000
H = 128
OUT = 16
R = 3
E = 200000

NC = 2
NS = 16
TILES = NC * NS

K = 128
FULL = E // K
TAIL = E - FULL * K
TAIL_BASE = FULL * K
CPT = -(-FULL // TILES)

WB = 200
NWB = N // WB
WPT = -(-NWB // NS)


def _make_sc_agg(h_dim):
    mesh = plsc.VectorSubcoreMesh(core_axis_name="c", subcore_axis_name="s")

    @functools.partial(
        pl.kernel,
        mesh=mesh,
        out_type=jax.ShapeDtypeStruct((NC, N, h_dim), jnp.float32),
        scratch_types=[
            pltpu.VMEM_SHARED((N, h_dim), jnp.float32),
            pltpu.VMEM((2, K), jnp.int32),
            pltpu.VMEM((K, h_dim), jnp.float32),
            pltpu.VMEM((2, TAIL), jnp.int32),
            pltpu.VMEM((TAIL, h_dim), jnp.float32),
            pltpu.VMEM((WB, h_dim), jnp.float32),
            pltpu.SemaphoreType.DMA,
        ],
    )
    def agg(t0, t1, t2, e0, e1, e2, zeros, out, acc, eidx, rows, eidx_t,
            rows_t, wbuf, sem):
        c = lax.axis_index("c")
        s = lax.axis_index("s")
        wid = c * NS + s

        pltpu.sync_copy(zeros, wbuf)
        for k in range(WPT):
            m = s + NS * k

            @pl.when(m < NWB)
            def _(m=m):
                r0 = pl.multiple_of(m * WB, WB)
                pltpu.sync_copy(wbuf, acc.at[pl.ds(r0, WB)])

        plsc.subcore_barrier()

        for tab, edg in ((t0, e0), (t1, e1), (t2, e2)):
            def step(j, _, tab=tab, edg=edg):
                ch = wid * CPT + j

                @pl.when(ch < FULL)
                def _():
                    base = ch * K
                    pltpu.sync_copy(edg.at[:, pl.ds(base, K)], eidx)
                    pltpu.async_copy(tab.at[eidx.at[0]], rows, sem).wait()
                    pltpu.sync_copy(rows, acc.at[eidx.at[1]], add=True)

                return 0

            lax.fori_loop(0, CPT, step, 0)

            @pl.when(wid == TILES - 1)
            def _(tab=tab, edg=edg):
                pltpu.sync_copy(edg.at[:, pl.ds(TAIL_BASE, TAIL)], eidx_t)
                pltpu.async_copy(tab.at[eidx_t.at[0]], rows_t, sem).wait()
                pltpu.sync_copy(rows_t, acc.at[eidx_t.at[1]], add=True)

        plsc.subcore_barrier()

        for k in range(WPT):
            m = s + NS * k

            @pl.when(m < NWB)
            def _(m=m):
                r0 = pl.multiple_of(m * WB, WB)
                pltpu.sync_copy(acc.at[pl.ds(r0, WB)], wbuf)
                pltpu.sync_copy(wbuf, out.at[c, pl.ds(r0, WB)])

    return agg


_sc_agg_h = _make_sc_agg(H)


BN = 400


def _tc_dense_body(p_ref, b_ref, w_ref, o_ref):
    h = jnp.maximum(p_ref[0] + p_ref[1] + b_ref[0], 0.0)
    for r in range(R):
        o_ref[r] = jnp.dot(h, w_ref[r], preferred_element_type=jnp.float32)


def _tc_dense(part, b, w):
    return pl.pallas_call(
        _tc_dense_body,
        grid=(N // BN,),
        in_specs=[
            pl.BlockSpec((NC, BN, H), lambda i: (0, i, 0)),
            pl.BlockSpec((1, H), lambda i: (0, 0)),
            pl.BlockSpec((R, H, H), lambda i: (0, 0, 0)),
        ],
        out_specs=pl.BlockSpec((R, BN, H), lambda i: (0, i, 0)),
        out_shape=jax.ShapeDtypeStruct((R, N, H), jnp.float32),
    )(part, b, w)


def _tc_final_body(p_ref, b_ref, o_ref):
    o_ref[...] = p_ref[0, :, :OUT] + p_ref[1, :, :OUT] + b_ref[0]


def _tc_final(part, b):
    return pl.pallas_call(
        _tc_final_body,
        grid=(N // BN,),
        in_specs=[
            pl.BlockSpec((NC, BN, H), lambda i: (0, i, 0)),
            pl.BlockSpec((1, OUT), lambda i: (0, 0)),
        ],
        out_specs=pl.BlockSpec((BN, OUT), lambda i: (i, 0)),
        out_shape=jax.ShapeDtypeStruct((N, OUT), jnp.float32),
    )(part, b)


@jax.jit
def kernel(embed, b0, w1, b1, w2, b2, edge_index_0, edge_index_1,
           edge_index_2):
    zeros_h = jnp.zeros((WB, H), jnp.float32)
    w2_pad = jnp.zeros((R, H, H), jnp.float32).at[:, :, :OUT].set(w2)
    e0, e1, e2 = edge_index_0, edge_index_1, edge_index_2

    p0 = _sc_agg_h(embed, embed, embed, e0, e1, e2, zeros_h)
    y = _tc_dense(p0, b0.reshape(1, H), w1)
    p1 = _sc_agg_h(y[0], y[1], y[2], e0, e1, e2, zeros_h)
    z = _tc_dense(p1, b1.reshape(1, H), w2_pad)
    p2 = _sc_agg_h(z[0], z[1], z[2], e0, e1, e2, zeros_h)
    return _tc_final(p2, b2.reshape(1, OUT))
